# Initial kernel scaffold; baseline (speedup 1.0000x reference)
import jax
import jax.numpy as jnp
from jax.experimental import pallas as pl


def kernel(X):
    raise NotImplementedError("write your pallas kernel here")


# speedup vs baseline: 5.3361x; 5.3361x over previous
import functools
import itertools as it

import jax
import jax.numpy as jnp
from jax import lax
from jax.experimental import pallas as pl
from jax.experimental.pallas import tpu as pltpu
from jax.experimental.pallas import tpu_sc as plsc

_K = 6
_SKIP = 4
_TRANSIENTS = 200
_P = 3

_B, _R, _T = 8, 16, 2048
_NROWS = _B * _R
_TOUT = _T - _TRANSIENTS
_NLIN = _K
_MONOMS = tuple(it.combinations_with_replacement(range(_NLIN), _P))
_NFEAT = 1 + _NLIN + len(_MONOMS)

_NWORKERS = 32
_ROWS_PER_W = _NROWS // _NWORKERS

_BLK = 16
_BLOCKS_PER_CHUNK = 29
_CHUNK_T = _BLOCKS_PER_CHUNK * _BLK
_NCHUNKS = 4
_XPAD = 2064
_STAGE = _CHUNK_T * _NFEAT
_OUTROW = _TOUT * _NFEAT


def _body(x_hbm, out_hbm, xin, stage):
    cid = lax.axis_index("c")
    sid = lax.axis_index("s")
    wid = sid * 2 + cid
    iota63 = lax.iota(jnp.int32, _BLK) * _NFEAT

    def row_body(rr, carry):
        r = wid * _ROWS_PER_W + rr
        pltpu.sync_copy(x_hbm.at[pl.ds(r * _XPAD, _XPAD)], xin)

        for c in range(_NCHUNKS):
            def blk(tb, carry2):
                t0 = c * _CHUNK_T + tb * _BLK
                lin = [xin[pl.ds(t0 + 180 + _SKIP * j, _BLK)] for j in range(_NLIN)]
                pairs = {}
                for a in range(_NLIN):
                    for b in range(a, _NLIN):
                        pairs[(a, b)] = lin[a] * lin[b]
                idxb = iota63 + tb * (_BLK * _NFEAT)
                ones = jnp.full((_BLK,), 1.0, dtype=jnp.float32)
                plsc.store_scatter(stage, [idxb], ones)
                for j in range(_NLIN):
                    plsc.store_scatter(stage, [idxb + (1 + j)], lin[j])
                for m, (i, j, k) in enumerate(_MONOMS):
                    plsc.store_scatter(stage, [idxb + (1 + _NLIN + m)],
                                       pairs[(i, j)] * lin[k])
                return carry2

            lax.fori_loop(0, _BLOCKS_PER_CHUNK, blk, 0)
            n_t = min(_CHUNK_T, _TOUT - c * _CHUNK_T)
            pltpu.sync_copy(
                stage.at[pl.ds(0, n_t * _NFEAT)],
                out_hbm.at[pl.ds(r * _OUTROW + c * _CHUNK_T * _NFEAT,
                                 n_t * _NFEAT)],
            )
        return carry

    lax.fori_loop(0, _ROWS_PER_W, row_body, 0)


@functools.partial(jax.jit)
def kernel(X):
    Xf = X.reshape(_NROWS, _T)
    Xf = jnp.pad(Xf, ((0, 0), (0, _XPAD - _T))).reshape(_NROWS * _XPAD)
    mesh = plsc.VectorSubcoreMesh(core_axis_name="c", subcore_axis_name="s")
    out = pl.kernel(
        _body,
        out_type=jax.ShapeDtypeStruct((_NROWS * _OUTROW,), jnp.float32),
        mesh=mesh,
        compiler_params=pltpu.CompilerParams(needs_layout_passes=False),
        scratch_types=[
            pltpu.VMEM((_XPAD,), jnp.float32),
            pltpu.VMEM((_STAGE,), jnp.float32),
        ],
    )(Xf)
    return out.reshape(_B, _R, _TOUT, _NFEAT)


# speedup vs baseline: 5.3381x; 1.0004x over previous
import functools
import itertools as it

import jax
import jax.numpy as jnp
from jax import lax
from jax.experimental import pallas as pl
from jax.experimental.pallas import tpu as pltpu
from jax.experimental.pallas import tpu_sc as plsc

_K = 6
_SKIP = 4
_TRANSIENTS = 200
_P = 3

_B, _R, _T = 8, 16, 2048
_NROWS = _B * _R
_TOUT = _T - _TRANSIENTS
_NLIN = _K
_MONOMS = tuple(it.combinations_with_replacement(range(_NLIN), _P))
_NFEAT = 1 + _NLIN + len(_MONOMS)

_NWORKERS = 32
_ROWS_PER_W = _NROWS // _NWORKERS

_BLK = 16
_BLOCKS_PER_CHUNK = 29
_CHUNK_T = _BLOCKS_PER_CHUNK * _BLK
_NCHUNKS = 4
_XPAD = 2064
_STAGE = _CHUNK_T * _NFEAT
_OUTROW = _TOUT * _NFEAT


def _body(x_hbm, out_hbm, xin, stage):
    cid = lax.axis_index("c")
    sid = lax.axis_index("s")
    wid = sid * 2 + cid
    iota63 = lax.iota(jnp.int32, _BLK) * _NFEAT

    def row_body(rr, carry):
        r = wid * _ROWS_PER_W + rr
        pltpu.sync_copy(x_hbm.at[pl.ds(r * _T, _T)], xin.at[pl.ds(0, _T)])

        for c in range(_NCHUNKS):
            def blk(tb, carry2):
                t0 = c * _CHUNK_T + tb * _BLK
                lin = [xin[pl.ds(t0 + 180 + _SKIP * j, _BLK)] for j in range(_NLIN)]
                pairs = {}
                for a in range(_NLIN):
                    for b in range(a, _NLIN):
                        pairs[(a, b)] = lin[a] * lin[b]
                idxb = iota63 + tb * (_BLK * _NFEAT)
                ones = jnp.full((_BLK,), 1.0, dtype=jnp.float32)
                plsc.store_scatter(stage, [idxb], ones)
                for j in range(_NLIN):
                    plsc.store_scatter(stage, [idxb + (1 + j)], lin[j])
                for m, (i, j, k) in enumerate(_MONOMS):
                    plsc.store_scatter(stage, [idxb + (1 + _NLIN + m)],
                                       pairs[(i, j)] * lin[k])
                return carry2

            lax.fori_loop(0, _BLOCKS_PER_CHUNK, blk, 0)
            n_t = min(_CHUNK_T, _TOUT - c * _CHUNK_T)
            pltpu.sync_copy(
                stage.at[pl.ds(0, n_t * _NFEAT)],
                out_hbm.at[pl.ds(r * _OUTROW + c * _CHUNK_T * _NFEAT,
                                 n_t * _NFEAT)],
            )
        return carry

    lax.fori_loop(0, _ROWS_PER_W, row_body, 0)


@functools.partial(jax.jit)
def kernel(X):
    Xf = X.reshape(_NROWS * _T)
    mesh = plsc.VectorSubcoreMesh(core_axis_name="c", subcore_axis_name="s")
    out = pl.kernel(
        _body,
        out_type=jax.ShapeDtypeStruct((_NROWS * _OUTROW,), jnp.float32),
        mesh=mesh,
        compiler_params=pltpu.CompilerParams(needs_layout_passes=False),
        scratch_types=[
            pltpu.VMEM((_XPAD,), jnp.float32),
            pltpu.VMEM((_STAGE,), jnp.float32),
        ],
    )(Xf)
    return out.reshape(_B, _R, _TOUT, _NFEAT)
